# baseline (speedup 1.0000x reference)
import functools

import jax
import jax.numpy as jnp
from jax import lax
from jax.experimental import pallas as pl
from jax.experimental.pallas import tpu as pltpu
from jax.experimental.pallas import tpu_sc as plsc

_L = 16


@functools.cache
def _build(B, HL, D, NE):
    info = plsc.get_sparse_core_info()
    NC, NS = info.num_cores, info.num_subcores
    NW = NC * NS
    assert B % NW == 0
    bags_per_w = B // NW
    idx_per_w = bags_per_w * HL
    CB = 32
    n_chunks = bags_per_w // CB
    idx_per_chunk = CB * HL
    n_vec = idx_per_w // _L

    mesh = plsc.VectorSubcoreMesh(core_axis_name="c", subcore_axis_name="s")

    @functools.partial(
        pl.kernel,
        out_type=(
            jax.ShapeDtypeStruct((B, D), jnp.float32),
            jax.ShapeDtypeStruct((B * HL,), jnp.int32),
        ),
        mesh=mesh,
        compiler_params=pltpu.CompilerParams(use_tc_tiling_on_sc=False),
        scratch_types=[
            pltpu.VMEM((idx_per_w,), jnp.int32),
            pltpu.VMEM((idx_per_w,), jnp.int32),
            pltpu.VMEM((idx_per_chunk, D), jnp.float32),
            pltpu.VMEM((CB, D), jnp.float32),
            pltpu.SemaphoreType.DMA,
        ],
    )
    def k(vals_hbm, table_hbm, out_hbm, remap_hbm, vals_v, idx_v, rows_v, out_v, sem):
        wid = lax.axis_index("s") * NC + lax.axis_index("c")
        base = wid * idx_per_w
        pltpu.sync_copy(vals_hbm.at[pl.ds(base, idx_per_w)], vals_v)

        def remap_body(j, _):
            v = vals_v[pl.ds(j * _L, _L)]
            idx_v[pl.ds(j * _L, _L)] = (v * 31 + 17) % NE
            return 0

        lax.fori_loop(0, n_vec, remap_body, 0)
        pltpu.sync_copy(idx_v, remap_hbm.at[pl.ds(base, idx_per_w)])

        def chunk_body(c, _):
            pltpu.async_copy(
                table_hbm.at[idx_v.at[pl.ds(c * idx_per_chunk, idx_per_chunk)]],
                rows_v, sem).wait()

            def bag_body(b, _):
                r0 = b * HL
                accs = [rows_v[r0, pl.ds(d * _L, _L)] for d in range(D // _L)]
                for l in range(1, HL):
                    for d in range(D // _L):
                        accs[d] += rows_v[r0 + l, pl.ds(d * _L, _L)]
                for d in range(D // _L):
                    out_v[b, pl.ds(d * _L, _L)] = accs[d]
                return 0

            lax.fori_loop(0, CB, bag_body, 0)
            pltpu.sync_copy(out_v, out_hbm.at[pl.ds(wid * bags_per_w + c * CB, CB)])
            return 0

        lax.fori_loop(0, n_chunks, chunk_body, 0)

    return k


def kernel(values, table):
    B, HL = values.shape
    NE, D = table.shape
    pooled, remap = _build(B, HL, D, NE)(values.reshape(-1), table)
    return pooled, remap.reshape(B, HL)
